# Initial kernel scaffold; baseline (speedup 1.0000x reference)
import jax
import jax.numpy as jnp
from jax.experimental import pallas as pl


def kernel(z, embedding_weight):
    raise NotImplementedError("write your pallas kernel here")


# speedup vs baseline: 1.2773x; 1.2773x over previous
import functools

import jax
import jax.numpy as jnp
from jax import lax
from jax.experimental import pallas as pl
from jax.experimental.pallas import tpu as pltpu
from jax.experimental.pallas import tpu_sc as plsc

_K = 8192
_D = 32
_ROWS = 16384
_RB = 512
_CB = 4096


def _vq_argmin_body(zq_ref, eq_ref, z2_ref, e2_ref, idx_ref):
    zq = zq_ref[...]
    z2 = z2_ref[...]

    def chunk(c, carry):
        run_min, run_idx = carry
        ec = eq_ref[pl.ds(c * _CB, _CB), :]
        e2 = e2_ref[:, pl.ds(c * _CB, _CB)]
        m = lax.dot_general(zq, ec, (((1,), (1,)), ((), ())),
                            preferred_element_type=jnp.float32)
        d = (z2 + e2) - 2.0 * m
        cmin = jnp.min(d, axis=1, keepdims=True)
        cols = lax.broadcasted_iota(jnp.int32, (_RB, _CB), 1) + c * _CB
        cidx = jnp.min(jnp.where(d == cmin, cols, _K), axis=1, keepdims=True)
        upd = cmin < run_min
        cmin_b = cmin.astype(jnp.bfloat16).astype(jnp.float32)
        return (jnp.where(upd, cmin_b, run_min), jnp.where(upd, cidx, run_idx))

    init = (jnp.full((_RB, 1), jnp.inf, jnp.float32),
            jnp.zeros((_RB, 1), jnp.int32))
    _, run_idx = lax.fori_loop(0, _K // _CB, chunk, init)
    idx_ref[...] = run_idx


_NW = 32
_BPW = _ROWS // _NW


@functools.cache
def _make_sc_gather():
    @functools.partial(
        pl.kernel,
        mesh=plsc.VectorSubcoreMesh(core_axis_name="c", subcore_axis_name="s"),
        out_type=jax.ShapeDtypeStruct((_ROWS, _D), jnp.float32),
        scratch_types=[
            pltpu.VMEM((_BPW,), jnp.int32),
            pltpu.VMEM((_BPW, _D), jnp.float32),
            pltpu.SemaphoreType.DMA,
        ],
        compiler_params=pltpu.CompilerParams(use_tc_tiling_on_sc=False),
    )
    def _sc_gather(table_hbm, idx_hbm, out_hbm, idx_v, rows_v, sem):
        wid = lax.axis_index("s") * 2 + lax.axis_index("c")
        base = wid * _BPW
        pltpu.sync_copy(idx_hbm.at[pl.ds(base, _BPW)], idx_v)
        pltpu.async_copy(table_hbm.at[idx_v], rows_v, sem).wait()
        pltpu.sync_copy(rows_v, out_hbm.at[pl.ds(base, _BPW)])

    return _sc_gather


def kernel(z, embedding_weight):
    B, C, H, W = z.shape
    zt = jnp.transpose(z, (0, 2, 3, 1)).reshape(-1, C)
    zq = zt.astype(jnp.bfloat16)
    eq = embedding_weight.astype(jnp.bfloat16)
    z2 = jnp.sum(zt ** 2, axis=1, keepdims=True)
    e2 = jnp.sum(embedding_weight ** 2, axis=1)[None, :]
    idx2 = pl.pallas_call(
        _vq_argmin_body,
        grid=(_ROWS // _RB,),
        in_specs=[
            pl.BlockSpec((_RB, _D), lambda i: (i, 0)),
            pl.BlockSpec((_K, _D), lambda i: (0, 0)),
            pl.BlockSpec((_RB, 1), lambda i: (i, 0)),
            pl.BlockSpec((1, _K), lambda i: (0, 0)),
        ],
        out_specs=pl.BlockSpec((_RB, 1), lambda i: (i, 0)),
        out_shape=jax.ShapeDtypeStruct((_ROWS, 1), jnp.int32),
    )(zq, eq, z2, e2)
    indices = idx2.reshape(-1)
    quantized = _make_sc_gather()(embedding_weight, indices).reshape(z.shape)
    return quantized, indices
